# baseline (speedup 1.0000x reference)
import functools

import jax
import jax.numpy as jnp
from jax import lax
from jax.experimental import pallas as pl
from jax.experimental.pallas import tpu as pltpu
from jax.experimental.pallas import tpu_sc as plsc

_K = 8
_M = 16
_LMASK = 127
_BLK = 2048
_IMAX = jnp.iinfo(jnp.int32).max
_QW = 32
_NW = 32


def _rank_body(n_real, xt_ref, q_ref, out_ref, dbest, ibest, work, flag):
    pid = pl.program_id(0)
    nblk = pl.num_programs(0)
    blk, dim = xt_ref.shape
    nq = q_ref.shape[1]

    @pl.when(pid == 0)
    def _init():
        dbest[...] = jnp.full((_M, nq), jnp.inf, jnp.float32)
        ibest[...] = jnp.full((_M, nq), _IMAX, jnp.int32)

    t = xt_ref[...]
    q = q_ref[...]
    t2 = jnp.sum(t * t, axis=1, keepdims=True)
    s = t2 - 2.0 * jax.lax.dot(t, q, precision=jax.lax.Precision.HIGHEST,
                               preferred_element_type=jnp.float32)

    gcol = pid * blk + lax.broadcasted_iota(jnp.int32, (blk, 1), 0)
    s = jnp.where(gcol < n_real, s, jnp.inf)

    work[...] = s
    flag[0] = 1

    def _insert(cd, cm):
        d_l = dbest[_M - 1:_M, :]
        m_l = ibest[_M - 1:_M, :]
        less = (cd < d_l) | ((cd == d_l) & (cm < m_l))
        dbest[_M - 1:_M, :] = jnp.where(less, cd, d_l)
        ibest[_M - 1:_M, :] = jnp.where(less, cm, m_l)
        for i in range(_M - 1, 0, -1):
            a_d = dbest[i:i + 1, :]
            a_m = ibest[i:i + 1, :]
            b_d = dbest[i - 1:i, :]
            b_m = ibest[i - 1:i, :]
            sw = (a_d < b_d) | ((a_d == b_d) & (a_m < b_m))
            dbest[i:i + 1, :] = jnp.where(sw, b_d, a_d)
            dbest[i - 1:i, :] = jnp.where(sw, a_d, b_d)
            ibest[i:i + 1, :] = jnp.where(sw, b_m, a_m)
            ibest[i - 1:i, :] = jnp.where(sw, a_m, b_m)

    for _ in range(_M):
        @pl.when(flag[0] == 1)
        def _pass():
            w = work[...]
            m = jnp.min(w, axis=0, keepdims=True)
            tau = dbest[_M - 1:_M, :]
            go = jnp.any(m < tau)
            flag[0] = go.astype(jnp.int32)

            @pl.when(go)
            def _extract():
                sel = jnp.min(jnp.where(w == m, gcol, _IMAX),
                              axis=0, keepdims=True)
                work[...] = jnp.where(gcol == sel, jnp.inf, w)
                _insert(m, sel)

    @pl.when(pid == nblk - 1)
    def _out():
        out_ref[...] = ibest[...].T


def _rank_call(n, dim, nq, nblk):
    return pl.pallas_call(
        functools.partial(_rank_body, n),
        grid=(nblk,),
        in_specs=[
            pl.BlockSpec((_BLK, dim), lambda i: (i, 0)),
            pl.BlockSpec((dim, nq), lambda i: (0, 0)),
        ],
        out_specs=pl.BlockSpec((nq, _M), lambda i: (0, 0)),
        out_shape=jax.ShapeDtypeStruct((nq, _M), jnp.int32),
        scratch_shapes=[
            pltpu.VMEM((_M, nq), jnp.float32),
            pltpu.VMEM((_M, nq), jnp.int32),
            pltpu.VMEM((_BLK, nq), jnp.float32),
            pltpu.SMEM((1,), jnp.int32),
        ],
    )


def _gather_body(dim, idx_hbm, xf_hbm, y_hbm, g_hbm, lab_hbm,
                 idx_v, ei_v, g_v, lab_v, sem):
    wid = lax.axis_index("s") * 2 + lax.axis_index("c")
    nper = _QW * _M
    base = wid * nper
    pltpu.sync_copy(idx_hbm.at[pl.ds(base, nper)], idx_v)
    for qq in range(_QW):
        r16 = idx_v[pl.ds(qq * _M, _M)] * dim
        for d in range(dim):
            ei_v[pl.ds((d * _QW + qq) * _M, _M)] = r16 + d
    copies = []
    for c in range(dim * nper // 128):
        copies.append(pltpu.async_copy(
            xf_hbm.at[ei_v.at[pl.ds(c * 128, 128)]],
            g_v.at[pl.ds(c * 128, 128)], sem))
    for c in range(nper // 128):
        copies.append(pltpu.async_copy(
            y_hbm.at[idx_v.at[pl.ds(c * 128, 128)]],
            lab_v.at[pl.ds(c * 128, 128)], sem))
    for cp in copies:
        cp.wait()
    for d in range(dim):
        pltpu.sync_copy(g_v.at[pl.ds(d * nper, nper)],
                        g_hbm.at[pl.ds((d * _NW + wid) * nper, nper)])
    pltpu.sync_copy(lab_v, lab_hbm.at[pl.ds(base, nper)])


def _gather_call(n, dim, nq):
    mesh = plsc.VectorSubcoreMesh(core_axis_name="c", subcore_axis_name="s")
    return pl.kernel(
        functools.partial(_gather_body, dim),
        out_type=(
            jax.ShapeDtypeStruct((dim * nq * _M,), jnp.float32),
            jax.ShapeDtypeStruct((nq * _M,), jnp.int32),
        ),
        mesh=mesh,
        scratch_types=[
            pltpu.VMEM((_QW * _M,), jnp.int32),
            pltpu.VMEM((16 * _QW * _M,), jnp.int32),
            pltpu.VMEM((16 * _QW * _M,), jnp.float32),
            pltpu.VMEM((_QW * _M,), jnp.int32),
            pltpu.SemaphoreType.DMA,
        ],
    )


def _refine_body(g_ref, lab_ref, idx_ref, q_ref, out_ref):
    dim, nq, m = g_ref.shape
    acc = jnp.zeros((nq, m), jnp.float32)
    for d in range(dim):
        diff = g_ref[d] - q_ref[:, d:d + 1]
        acc = acc + diff * diff
    w = jnp.sqrt(acc)
    idxs = idx_ref[...]
    labs = lab_ref[...]

    lab8 = []
    for _ in range(_K):
        mn = jnp.min(w, axis=1, keepdims=True)
        sel = jnp.min(jnp.where(w == mn, idxs, _IMAX), axis=1, keepdims=True)
        lsel = jnp.min(jnp.where(idxs == sel, labs, _IMAX),
                       axis=1, keepdims=True)
        w = jnp.where(idxs == sel, jnp.inf, w)
        lab8.append(lsel)
    lab8 = jnp.concatenate(lab8, axis=1)
    cnt = jnp.zeros((nq, _K), jnp.int32)
    for j in range(_K):
        cnt = cnt + (lab8 == lab8[:, j:j + 1]).astype(jnp.int32)
    key = (_K - cnt) * (_LMASK + 1) + lab8
    best = jnp.min(key, axis=1, keepdims=True)
    out_ref[...] = best & _LMASK


def _refine_call(dim, nq):
    return pl.pallas_call(
        _refine_body,
        out_shape=jax.ShapeDtypeStruct((nq, 1), jnp.int32),
    )


def kernel(X_test, X_train, y_train):
    n, dim = X_train.shape
    nq = X_test.shape[0]
    nblk = -(-n // _BLK)
    npad = nblk * _BLK
    xt = jnp.pad(X_train, ((0, npad - n), (0, 0)))
    qt = X_test.T
    idx16 = _rank_call(n, dim, nq, nblk)(xt, qt)
    g, lab = _gather_call(n, dim, nq)(
        idx16.reshape(nq * _M), X_train.reshape(n * dim),
        y_train.astype(jnp.int32))
    pred = _refine_call(dim, nq)(
        g.reshape(dim, nq, _M), lab.reshape(nq, _M), idx16, X_test)
    return pred.reshape(nq)


# speedup vs baseline: 2.5225x; 2.5225x over previous
import functools

import jax
import jax.numpy as jnp
from jax import lax
from jax.experimental import pallas as pl
from jax.experimental.pallas import tpu as pltpu
from jax.experimental.pallas import tpu_sc as plsc

_K = 8
_M = 12
_LMASK = 127
_BLK = 512
_IMAX = jnp.iinfo(jnp.int32).max
_QW = 32
_NW = 32


def _rank_body(n_real, xt_ref, q_ref, out_ref, dbest, ibest, work, flag):
    pid = pl.program_id(0)
    nblk = pl.num_programs(0)
    blk, dim = xt_ref.shape
    nq = q_ref.shape[1]

    @pl.when(pid == 0)
    def _init():
        dbest[...] = jnp.full((_M, nq), jnp.inf, jnp.float32)
        ibest[...] = jnp.full((_M, nq), _IMAX, jnp.int32)

    t = xt_ref[...]
    q = q_ref[...]
    t2 = jnp.sum(t * t, axis=1, keepdims=True)
    s = t2 - 2.0 * jax.lax.dot(t, q, precision=jax.lax.Precision.HIGHEST,
                               preferred_element_type=jnp.float32)

    gcol = pid * blk + lax.broadcasted_iota(jnp.int32, (blk, 1), 0)
    s = jnp.where(gcol < n_real, s, jnp.inf)

    work[...] = s
    flag[0] = 1

    def _insert(cd, cm):
        d_l = dbest[_M - 1:_M, :]
        m_l = ibest[_M - 1:_M, :]
        less = (cd < d_l) | ((cd == d_l) & (cm < m_l))
        dbest[_M - 1:_M, :] = jnp.where(less, cd, d_l)
        ibest[_M - 1:_M, :] = jnp.where(less, cm, m_l)
        for i in range(_M - 1, 0, -1):
            a_d = dbest[i:i + 1, :]
            a_m = ibest[i:i + 1, :]
            b_d = dbest[i - 1:i, :]
            b_m = ibest[i - 1:i, :]
            sw = (a_d < b_d) | ((a_d == b_d) & (a_m < b_m))
            dbest[i:i + 1, :] = jnp.where(sw, b_d, a_d)
            dbest[i - 1:i, :] = jnp.where(sw, a_d, b_d)
            ibest[i:i + 1, :] = jnp.where(sw, b_m, a_m)
            ibest[i - 1:i, :] = jnp.where(sw, a_m, b_m)

    for _ in range(_M):
        @pl.when(flag[0] == 1)
        def _pass():
            w = work[...]
            m = jnp.min(w, axis=0, keepdims=True)
            tau = dbest[_M - 1:_M, :]
            go = jnp.any(m < tau)
            flag[0] = go.astype(jnp.int32)

            @pl.when(go)
            def _extract():
                sel = jnp.min(jnp.where(w == m, gcol, _IMAX),
                              axis=0, keepdims=True)
                work[...] = jnp.where(gcol == sel, jnp.inf, w)
                _insert(m, sel)

    @pl.when(pid == nblk - 1)
    def _out():
        out_ref[...] = ibest[...].T


def _rank_call(n, dim, nq, nblk):
    return pl.pallas_call(
        functools.partial(_rank_body, n),
        grid=(nblk,),
        in_specs=[
            pl.BlockSpec((_BLK, dim), lambda i: (i, 0)),
            pl.BlockSpec((dim, nq), lambda i: (0, 0)),
        ],
        out_specs=pl.BlockSpec((nq, _M), lambda i: (0, 0)),
        out_shape=jax.ShapeDtypeStruct((nq, _M), jnp.int32),
        scratch_shapes=[
            pltpu.VMEM((_M, nq), jnp.float32),
            pltpu.VMEM((_M, nq), jnp.int32),
            pltpu.VMEM((_BLK, nq), jnp.float32),
            pltpu.SMEM((1,), jnp.int32),
        ],
    )


def _gather_body(dim, idx_hbm, xf_hbm, y_hbm, g_hbm, lab_hbm,
                 idx_v, ei_v, g_v, lab_v, sem):
    wid = lax.axis_index("s") * 2 + lax.axis_index("c")
    nper = _QW * _M
    base = wid * nper
    pltpu.sync_copy(idx_hbm.at[pl.ds(base, nper)], idx_v)
    for qq in range(_QW):
        r16 = idx_v[pl.ds(qq * _M, _M)] * dim
        for d in range(dim):
            ei_v[pl.ds((d * _QW + qq) * _M, _M)] = r16 + d
    copies = []
    for c in range(dim * nper // 128):
        copies.append(pltpu.async_copy(
            xf_hbm.at[ei_v.at[pl.ds(c * 128, 128)]],
            g_v.at[pl.ds(c * 128, 128)], sem))
    for c in range(nper // 128):
        copies.append(pltpu.async_copy(
            y_hbm.at[idx_v.at[pl.ds(c * 128, 128)]],
            lab_v.at[pl.ds(c * 128, 128)], sem))
    for cp in copies:
        cp.wait()
    for d in range(dim):
        pltpu.sync_copy(g_v.at[pl.ds(d * nper, nper)],
                        g_hbm.at[pl.ds((d * _NW + wid) * nper, nper)])
    pltpu.sync_copy(lab_v, lab_hbm.at[pl.ds(base, nper)])


def _gather_call(n, dim, nq):
    mesh = plsc.VectorSubcoreMesh(core_axis_name="c", subcore_axis_name="s")
    return pl.kernel(
        functools.partial(_gather_body, dim),
        out_type=(
            jax.ShapeDtypeStruct((dim * nq * _M,), jnp.float32),
            jax.ShapeDtypeStruct((nq * _M,), jnp.int32),
        ),
        mesh=mesh,
        scratch_types=[
            pltpu.VMEM((_QW * _M,), jnp.int32),
            pltpu.VMEM((16 * _QW * _M,), jnp.int32),
            pltpu.VMEM((16 * _QW * _M,), jnp.float32),
            pltpu.VMEM((_QW * _M,), jnp.int32),
            pltpu.SemaphoreType.DMA,
        ],
    )


def _refine_body(g_ref, lab_ref, idx_ref, q_ref, out_ref):
    dim, nq, m = g_ref.shape
    acc = jnp.zeros((nq, m), jnp.float32)
    for d in range(dim):
        diff = g_ref[d] - q_ref[:, d:d + 1]
        acc = acc + diff * diff
    w = jnp.sqrt(acc)
    idxs = idx_ref[...]
    labs = lab_ref[...]

    lab8 = []
    for _ in range(_K):
        mn = jnp.min(w, axis=1, keepdims=True)
        sel = jnp.min(jnp.where(w == mn, idxs, _IMAX), axis=1, keepdims=True)
        lsel = jnp.min(jnp.where(idxs == sel, labs, _IMAX),
                       axis=1, keepdims=True)
        w = jnp.where(idxs == sel, jnp.inf, w)
        lab8.append(lsel)
    lab8 = jnp.concatenate(lab8, axis=1)
    cnt = jnp.zeros((nq, _K), jnp.int32)
    for j in range(_K):
        cnt = cnt + (lab8 == lab8[:, j:j + 1]).astype(jnp.int32)
    key = (_K - cnt) * (_LMASK + 1) + lab8
    best = jnp.min(key, axis=1, keepdims=True)
    out_ref[...] = best & _LMASK


def _refine_call(dim, nq):
    return pl.pallas_call(
        _refine_body,
        out_shape=jax.ShapeDtypeStruct((nq, 1), jnp.int32),
    )


def kernel(X_test, X_train, y_train):
    n, dim = X_train.shape
    nq = X_test.shape[0]
    nblk = -(-n // _BLK)
    npad = nblk * _BLK
    xt = jnp.pad(X_train, ((0, npad - n), (0, 0)))
    qt = X_test.T
    idx16 = _rank_call(n, dim, nq, nblk)(xt, qt)
    g, lab = _gather_call(n, dim, nq)(
        idx16.reshape(nq * _M), X_train.reshape(n * dim),
        y_train.astype(jnp.int32))
    pred = _refine_call(dim, nq)(
        g.reshape(dim, nq, _M), lab.reshape(nq, _M), idx16, X_test)
    return pred.reshape(nq)
